# Initial kernel scaffold; baseline (speedup 1.0000x reference)
import jax
import jax.numpy as jnp
from jax.experimental import pallas as pl


def kernel(features, edge_index, W1_self, W1_neigh, b1, W2_self, W2_neigh, b2):
    raise NotImplementedError("write your pallas kernel here")


# speedup vs baseline: 13.6554x; 13.6554x over previous
import functools

import jax
import jax.numpy as jnp
from jax import lax
from jax.experimental import pallas as pl
from jax.experimental.pallas import tpu as pltpu
from jax.experimental.pallas import tpu_sc as plsc


_LANE = 128
_KCH = 23
_RPW = 391


def _build_sc_agg(N, N_pad, R_pad, table_width):
    info = plsc.get_sparse_core_info()
    NC, NS = info.num_cores, info.num_subcores
    NW = NC * NS
    assert R_pad == NW * _RPW and _RPW % _KCH == 0
    RPT = N_pad // NS
    assert N_pad % NS == 0 and (RPT * table_width) % 8 == 0

    mesh = plsc.VectorSubcoreMesh(core_axis_name="c", subcore_axis_name="s")

    @functools.partial(
        pl.kernel,
        mesh=mesh,
        compiler_params=pltpu.CompilerParams(use_tc_tiling_on_sc=False),
        out_type=jax.ShapeDtypeStruct((NC, N_pad, table_width), jnp.float32),
        scratch_types=[
            pltpu.VMEM((_KCH, _LANE), jnp.int32),
            pltpu.VMEM((_KCH, _LANE), jnp.int32),
            pltpu.VMEM((_LANE, table_width), jnp.float32),
            pltpu.VMEM_SHARED((N_pad, table_width), jnp.float32),
            pltpu.SemaphoreType.DMA,
        ],
    )
    def sc_agg(src_hbm, dst_hbm, table_hbm, zeros_hbm, out_hbm,
               src_v, dst_v, rows_v, acc_sh, sem):
        c = lax.axis_index("c")
        s = lax.axis_index("s")
        w = c * NS + s
        z0 = s * RPT
        pltpu.sync_copy(zeros_hbm.at[pl.ds(z0, RPT)], acc_sh.at[pl.ds(z0, RPT)])
        plsc.subcore_barrier()
        base_row = w * _RPW

        def inner(j, carry):
            pltpu.async_copy(table_hbm.at[src_v.at[j]], rows_v, sem).wait()
            pltpu.sync_copy(rows_v, acc_sh.at[dst_v.at[j]], add=True)
            return carry

        def outer(g, carry):
            r0 = base_row + g * _KCH
            pltpu.sync_copy(src_hbm.at[pl.ds(r0, _KCH)], src_v)
            pltpu.sync_copy(dst_hbm.at[pl.ds(r0, _KCH)], dst_v)
            lax.fori_loop(0, _KCH, inner, 0)
            return carry

        lax.fori_loop(0, _RPW // _KCH, outer, 0)
        plsc.subcore_barrier()
        pltpu.sync_copy(acc_sh.at[pl.ds(z0, RPT)], out_hbm.at[c, pl.ds(z0, RPT)])

    return sc_agg


def _leaky(x):
    return jnp.where(x >= 0, x, 0.01 * x)


def _build_stage_b(N, N_pad, BN):
    G = N // BN
    assert N % BN == 0

    def body(x_ref, s1_ref, wcat_ref, b1_ref, w2s_ref, b2_ref, w2n_ref,
             p2n_ref, p2s_ref, inv_ref):
        x = x_ref[...]
        s1 = s1_ref[0] + s1_ref[1]
        inv = 1.0 / jnp.maximum(s1[:, 3:4], 1.0)
        m1 = s1[:, 0:3] * inv
        xm = jnp.concatenate([x, m1], axis=1)
        h = jnp.dot(xm, wcat_ref[...], preferred_element_type=jnp.float32)
        h = _leaky(h + b1_ref[...])
        p2n_ref[...] = jnp.dot(h, w2n_ref[...], preferred_element_type=jnp.float32)
        p2s_ref[...] = jnp.dot(h, w2s_ref[...], preferred_element_type=jnp.float32) + b2_ref[...]
        inv_ref[...] = inv

    wspec = lambda shape: pl.BlockSpec(shape, lambda i: (0,) * len(shape))
    return pl.pallas_call(
        body,
        grid=(G,),
        in_specs=[
            pl.BlockSpec((BN, 3), lambda i: (i, 0)),
            pl.BlockSpec((2, BN, 8), lambda i: (0, i, 0)),
            wspec((6, 32)),
            wspec((1, 32)),
            wspec((32, 3)),
            wspec((1, 3)),
            wspec((32, 8)),
        ],
        out_specs=[
            pl.BlockSpec((BN, 8), lambda i: (i, 0)),
            pl.BlockSpec((BN, 3), lambda i: (i, 0)),
            pl.BlockSpec((BN, 1), lambda i: (i, 0)),
        ],
        out_shape=[
            jax.ShapeDtypeStruct((N, 8), jnp.float32),
            jax.ShapeDtypeStruct((N, 3), jnp.float32),
            jax.ShapeDtypeStruct((N, 1), jnp.float32),
        ],
    )


def _build_stage_d(N, N_pad, BN):
    G = N // BN

    def body(p2s_ref, s2_ref, inv_ref, out_ref):
        s2 = s2_ref[0] + s2_ref[1]
        y = p2s_ref[...] + s2[:, 0:3] * inv_ref[...]
        out_ref[...] = _leaky(y)

    return pl.pallas_call(
        body,
        grid=(G,),
        in_specs=[
            pl.BlockSpec((BN, 3), lambda i: (i, 0)),
            pl.BlockSpec((2, BN, 8), lambda i: (0, i, 0)),
            pl.BlockSpec((BN, 1), lambda i: (i, 0)),
        ],
        out_specs=pl.BlockSpec((BN, 3), lambda i: (i, 0)),
        out_shape=jax.ShapeDtypeStruct((N, 3), jnp.float32),
    )


def kernel(features, edge_index, W1_self, W1_neigh, b1, W2_self, W2_neigh, b2):
    N = features.shape[0]
    E = edge_index.shape[1]
    N_pad = ((N + 127) // 128) * 128

    info = plsc.get_sparse_core_info()
    NW = info.num_cores * info.num_subcores
    R_pad = NW * _RPW
    E_pad = R_pad * _LANE
    assert E_pad >= E

    ei = edge_index.astype(jnp.int32)
    pad = jnp.full((2, E_pad - E), N, jnp.int32)
    ei = jnp.concatenate([ei, pad], axis=1)
    src = ei[0].reshape(R_pad, _LANE)
    dst = ei[1].reshape(R_pad, _LANE)

    feat8 = jnp.concatenate([features, jnp.ones((N, 1), jnp.float32)], axis=1)
    feat8 = jnp.pad(feat8, ((0, 8), (0, 4)))
    zeros = jnp.zeros((N_pad, 8), jnp.float32)

    sc_agg = _build_sc_agg(N, N_pad, R_pad, 8)
    sums1 = sc_agg(src, dst, feat8, zeros)

    w_cat = jnp.concatenate([W1_self.T, W1_neigh.T], axis=0)
    w2n_t8 = jnp.pad(W2_neigh, ((0, 5), (0, 0))).T

    BN = 1000
    stage_b = _build_stage_b(N, N_pad, BN)
    p2n8, p2s, inv = stage_b(features, sums1, w_cat, b1.reshape(1, 32),
                             W2_self.T, b2.reshape(1, 3), w2n_t8)

    sums2 = sc_agg(src, dst, jnp.pad(p2n8, ((0, 8), (0, 0))), zeros)

    stage_d = _build_stage_d(N, N_pad, BN)
    return stage_d(p2s, sums2, inv)


# speedup vs baseline: 21.2889x; 1.5590x over previous
import functools

import jax
import jax.numpy as jnp
from jax import lax
from jax.experimental import pallas as pl
from jax.experimental.pallas import tpu as pltpu
from jax.experimental.pallas import tpu_sc as plsc


_LANE = 128
_KCH = 23
_RPW = 391


def _build_sc_agg(N, N_pad, R_pad, table_width):
    info = plsc.get_sparse_core_info()
    NC, NS = info.num_cores, info.num_subcores
    NW = NC * NS
    assert R_pad == NW * _RPW and _RPW % _KCH == 0
    RPT = N_pad // NS
    assert N_pad % NS == 0 and (RPT * table_width) % 8 == 0

    mesh = plsc.VectorSubcoreMesh(core_axis_name="c", subcore_axis_name="s")

    @functools.partial(
        pl.kernel,
        mesh=mesh,
        compiler_params=pltpu.CompilerParams(use_tc_tiling_on_sc=False),
        out_type=jax.ShapeDtypeStruct((NC, N_pad, table_width), jnp.float32),
        scratch_types=[
            pltpu.VMEM((2, _KCH, _LANE), jnp.int32),
            pltpu.VMEM((2, _KCH, _LANE), jnp.int32),
            pltpu.VMEM((3, _LANE, table_width), jnp.float32),
            pltpu.VMEM_SHARED((N_pad, table_width), jnp.float32),
            pltpu.SemaphoreType.DMA,
            pltpu.SemaphoreType.DMA,
            pltpu.SemaphoreType.DMA,
            pltpu.SemaphoreType.DMA,
        ],
    )
    def sc_agg(src_hbm, dst_hbm, table_hbm, zeros_hbm, out_hbm,
               src_v, dst_v, rows_v, acc_sh, isem, gsem0, gsem1, gsem2):
        c = lax.axis_index("c")
        s = lax.axis_index("s")
        w = c * NS + s
        z0 = s * RPT
        pltpu.sync_copy(zeros_hbm.at[pl.ds(z0, RPT)], acc_sh.at[pl.ds(z0, RPT)])
        plsc.subcore_barrier()
        base_row = w * _RPW
        NCH = _RPW // _KCH
        gsems = (gsem0, gsem1, gsem2)

        def stage_idx(g, bank):
            r0 = base_row + g * _KCH
            pltpu.async_copy(src_hbm.at[pl.ds(r0, _KCH)], src_v.at[bank], isem)
            pltpu.async_copy(dst_hbm.at[pl.ds(r0, _KCH)], dst_v.at[bank], isem)

        stage_idx(0, 0)

        def outer(g, carry):
            gb = g % 2
            pltpu.make_async_copy(src_hbm.at[pl.ds(base_row, _KCH)],
                                  src_v.at[gb], isem).wait()
            pltpu.make_async_copy(dst_hbm.at[pl.ds(base_row, _KCH)],
                                  dst_v.at[gb], isem).wait()

            @pl.when(g < NCH - 1)
            def _():
                stage_idx(g + 1, 1 - gb)

            hs = {}
            hs[0] = pltpu.async_copy(table_hbm.at[src_v.at[gb, 0]],
                                     rows_v.at[0], gsems[0])
            hs[1] = pltpu.async_copy(table_hbm.at[src_v.at[gb, 1]],
                                     rows_v.at[1], gsems[1])
            for j in range(_KCH):
                b = j % 3
                if j + 2 < _KCH:
                    nb = (j + 2) % 3
                    hs[nb] = pltpu.async_copy(table_hbm.at[src_v.at[gb, j + 2]],
                                              rows_v.at[nb], gsems[nb])
                hs[b].wait()
                pltpu.sync_copy(rows_v.at[b], acc_sh.at[dst_v.at[gb, j]], add=True)
            return carry

        lax.fori_loop(0, NCH, outer, 0)
        plsc.subcore_barrier()
        pltpu.sync_copy(acc_sh.at[pl.ds(z0, RPT)], out_hbm.at[c, pl.ds(z0, RPT)])

    return sc_agg


def _leaky(x):
    return jnp.where(x >= 0, x, 0.01 * x)


def _build_stage_b(N, N_pad, BN):
    G = N // BN
    assert N % BN == 0

    def body(x_ref, s1_ref, wcat_ref, b1_ref, w2s_ref, b2_ref, w2n_ref,
             p2n_ref, p2s_ref, inv_ref):
        x = x_ref[...]
        s1 = s1_ref[0] + s1_ref[1]
        inv = 1.0 / jnp.maximum(s1[:, 3:4], 1.0)
        m1 = s1[:, 0:3] * inv
        xm = jnp.concatenate([x, m1], axis=1)
        h = jnp.dot(xm, wcat_ref[...], preferred_element_type=jnp.float32)
        h = _leaky(h + b1_ref[...])
        p2n_ref[...] = jnp.dot(h, w2n_ref[...], preferred_element_type=jnp.float32)
        p2s_ref[...] = jnp.dot(h, w2s_ref[...], preferred_element_type=jnp.float32) + b2_ref[...]
        inv_ref[...] = inv

    wspec = lambda shape: pl.BlockSpec(shape, lambda i: (0,) * len(shape))
    return pl.pallas_call(
        body,
        grid=(G,),
        in_specs=[
            pl.BlockSpec((BN, 3), lambda i: (i, 0)),
            pl.BlockSpec((2, BN, 8), lambda i: (0, i, 0)),
            wspec((6, 32)),
            wspec((1, 32)),
            wspec((32, 3)),
            wspec((1, 3)),
            wspec((32, 8)),
        ],
        out_specs=[
            pl.BlockSpec((BN, 8), lambda i: (i, 0)),
            pl.BlockSpec((BN, 3), lambda i: (i, 0)),
            pl.BlockSpec((BN, 1), lambda i: (i, 0)),
        ],
        out_shape=[
            jax.ShapeDtypeStruct((N + 8, 8), jnp.float32),
            jax.ShapeDtypeStruct((N, 3), jnp.float32),
            jax.ShapeDtypeStruct((N, 1), jnp.float32),
        ],
    )


def _build_stage_d(N, N_pad, BN):
    G = N // BN

    def body(p2s_ref, s2_ref, inv_ref, out_ref):
        s2 = s2_ref[0] + s2_ref[1]
        y = p2s_ref[...] + s2[:, 0:3] * inv_ref[...]
        out_ref[...] = _leaky(y)

    return pl.pallas_call(
        body,
        grid=(G,),
        in_specs=[
            pl.BlockSpec((BN, 3), lambda i: (i, 0)),
            pl.BlockSpec((2, BN, 8), lambda i: (0, i, 0)),
            pl.BlockSpec((BN, 1), lambda i: (i, 0)),
        ],
        out_specs=pl.BlockSpec((BN, 3), lambda i: (i, 0)),
        out_shape=jax.ShapeDtypeStruct((N, 3), jnp.float32),
    )


def kernel(features, edge_index, W1_self, W1_neigh, b1, W2_self, W2_neigh, b2):
    N = features.shape[0]
    E = edge_index.shape[1]
    N_pad = ((N + 127) // 128) * 128

    info = plsc.get_sparse_core_info()
    NW = info.num_cores * info.num_subcores
    R_pad = NW * _RPW
    E_pad = R_pad * _LANE
    assert E_pad >= E

    ei = jnp.pad(edge_index.astype(jnp.int32), ((0, 0), (0, E_pad - E)),
                 constant_values=N).reshape(2, R_pad, _LANE)
    src = ei[0]
    dst = ei[1]

    feat8 = jnp.concatenate([features, jnp.ones((N, 1), jnp.float32)], axis=1)
    feat8 = jnp.pad(feat8, ((0, 8), (0, 4)))
    zeros = jnp.zeros((N_pad, 8), jnp.float32)

    sc_agg = _build_sc_agg(N, N_pad, R_pad, 8)
    sums1 = sc_agg(src, dst, feat8, zeros)

    w_cat = jnp.concatenate([W1_self.T, W1_neigh.T], axis=0)
    w2n_t8 = jnp.pad(W2_neigh, ((0, 5), (0, 0))).T

    BN = 1000
    stage_b = _build_stage_b(N, N_pad, BN)
    p2n8, p2s, inv = stage_b(features, sums1, w_cat, b1.reshape(1, 32),
                             W2_self.T, b2.reshape(1, 3), w2n_t8)

    sums2 = sc_agg(src, dst, p2n8, zeros)

    stage_d = _build_stage_d(N, N_pad, BN)
    return stage_d(p2s, sums2, inv)
